# baseline (speedup 1.0000x reference)
import math

import jax
import jax.numpy as jnp
import numpy as np
from jax.experimental import pallas as pl
from jax.experimental.pallas import tpu as pltpu

_M, _D, _N, _R = 16384, 64, 8, 8
_T, _B = 8, 8
_S = 0.5
_PROBS = [0.7, 0.3, 0.5]
_WIN = 2 * _R + 1


def _ring_plan():
    step = int(_M * ((math.sqrt(5) - 1) / 2))
    ptr = np.zeros(_N, np.float32)
    centers = np.zeros((_T, _N), np.int64)
    for t in range(_T):
        for i in range(_N):
            c = int(np.clip(np.int32(ptr[i]), 0, _M - 1))
            centers[t, i] = c
            jump = np.float32((c + step + i) % _M)
            walk = np.float32((ptr[i] + np.float32(1.0)) % _M)
            p = np.float32(_PROBS[i % 3])
            q = np.float32(1.0 - _PROBS[i % 3])
            ptr[i] = np.float32(p * jump) + np.float32(q * walk)
    offs = np.arange(-_R, _R + 1)
    wins = (centers[:, :, None] + offs[None, None, :]) % _M

    gslots = []
    for g in range(3):
        s = np.unique(wins[1:, [i for i in range(_N) if i % 3 == g]])
        gslots.append({int(v): k for k, v in enumerate(s)})
    sizes = tuple(len(d) for d in gslots)

    segs = []
    for t in range(1, _T):
        row = []
        for i in range(_N):
            lut = gslots[i % 3]
            ks = sorted(lut[int(s)] for s in wins[t, i])
            runs = []
            a = prev = ks[0]
            for k in ks[1:]:
                if k == prev + 1:
                    prev = k
                else:
                    runs.append((a, prev - a + 1))
                    a = prev = k
            runs.append((a, prev - a + 1))
            row.append(runs)
        segs.append(row)
    return sizes, segs


_KS, _SEGS = _ring_plan()
_DN = (((1,), (1,)), ((), ()))


def _body(x_ref, wi_ref, bi_ref, wr_ref, br_ref, wo_ref, bo_ref,
          out_ref, ring0_ref, ring1_ref, ring2_ref):
    f32 = jnp.float32
    rings = (ring0_ref, ring1_ref, ring2_ref)
    for g in range(3):
        rings[g][:] = jnp.zeros((_KS[g], _B, _D), f32)
    wi = wi_ref[:]
    wo = wo_ref[:]
    bi = bi_ref[:]
    bo = bo_ref[:]

    def inp_at(t):
        xs = x_ref[:, t * _D:(t + 1) * _D]
        return jax.lax.dot_general(xs, wi, _DN,
                                   preferred_element_type=f32) + bi

    def emit_out(t, hidden):
        hs = hidden[0]
        for i in range(1, _N):
            hs = hs + hidden[i]
        res = jax.lax.dot_general(hs * (1.0 / _N), wo, _DN,
                                  preferred_element_type=f32) + bo
        out_ref[:, t * _D:(t + 1) * _D] = res

    inp0 = inp_at(0)
    hidden = []
    hpre = None
    for i in range(_N):
        if i == 0:
            h = inp0 + _S * br_ref[0:1, :]
        else:
            rv = jax.lax.dot_general(hpre * (1.0 / _WIN), wr_ref[i], _DN,
                                     preferred_element_type=f32)
            h = inp0 + _S * (rv + br_ref[i:i + 1, :])
        hidden.append(h)
        hpre = h if i == 0 else hpre + h
    emit_out(0, hidden)

    for t in range(1, _T):
        inp_t = inp_at(t)
        for i in range(_N):
            ring = rings[i % 3]
            h = hidden[i] + inp_t
            acc = None
            for (a, ln) in _SEGS[t - 1][i]:
                s = jnp.sum(ring[a:a + ln], axis=0)
                acc = s if acc is None else acc + s
            rv = jax.lax.dot_general(acc * (1.0 / _WIN), wr_ref[i], _DN,
                                     preferred_element_type=f32)
            h = h + _S * (rv + br_ref[i:i + 1, :])
            hidden[i] = h
            v = h * (1.0 / _WIN)
            for (a, ln) in _SEGS[t - 1][i]:
                ring[a:a + ln] = ring[a:a + ln] + v[None]
        emit_out(t, hidden)


def kernel(x, W_inp, b_inp, W_out, b_out, W_read, b_read):
    bb, tt, feat = x.shape
    xf = x.reshape(bb, tt * feat)
    out = pl.pallas_call(
        _body,
        out_shape=jax.ShapeDtypeStruct((bb, tt * feat), jnp.float32),
        scratch_shapes=[pltpu.VMEM((k, _B, _D), jnp.float32) for k in _KS],
    )(xf, W_inp, b_inp.reshape(1, -1), W_read, b_read,
      W_out, b_out.reshape(1, -1))
    return out.reshape(bb, tt, feat)


# speedup vs baseline: 1.1874x; 1.1874x over previous
import math

import jax
import jax.numpy as jnp
import numpy as np
from jax.experimental import pallas as pl
from jax.experimental.pallas import tpu as pltpu

_M, _D, _N, _R = 16384, 64, 8, 8
_T, _B = 8, 8
_S = 0.5
_PROBS = [0.7, 0.3, 0.5]
_WIN = 2 * _R + 1


def _coef_plan():
    step = int(_M * ((math.sqrt(5) - 1) / 2))
    ptr = np.zeros(_N, np.float32)
    centers = np.zeros((_T, _N), np.int64)
    for t in range(_T):
        for i in range(_N):
            c = int(np.clip(np.int32(ptr[i]), 0, _M - 1))
            centers[t, i] = c
            jump = np.float32((c + step + i) % _M)
            walk = np.float32((ptr[i] + np.float32(1.0)) % _M)
            p = np.float32(_PROBS[i % 3])
            q = np.float32(1.0 - _PROBS[i % 3])
            ptr[i] = np.float32(p * jump) + np.float32(q * walk)
    offs = np.arange(-_R, _R + 1)
    wins = (centers[:, :, None] + offs[None, None, :]) % _M

    coef = []
    for t in range(1, _T):
        row = []
        for i in range(_N):
            wi = set(wins[t, i].tolist())
            cs = []
            for j in range(i):
                c = len(wi & set(wins[t, j].tolist()))
                if c:
                    cs.append((j, c))
            row.append(cs)
        coef.append(row)
    return coef


_COEF = _coef_plan()
_DN = (((1,), (1,)), ((), ()))


def _body(x_ref, wi_ref, bi_ref, wr_ref, br_ref, wo_ref, bo_ref, out_ref):
    f32 = jnp.float32
    wi = wi_ref[:]
    wo = wo_ref[:]
    bi = bi_ref[:]
    bo = bo_ref[:]

    def inp_at(t):
        xs = x_ref[:, t * _D:(t + 1) * _D]
        return jax.lax.dot_general(xs, wi, _DN,
                                   preferred_element_type=f32) + bi

    def emit_out(t, hidden):
        hs = hidden[0]
        for i in range(1, _N):
            hs = hs + hidden[i]
        res = jax.lax.dot_general(hs * (1.0 / _N), wo, _DN,
                                  preferred_element_type=f32) + bo
        out_ref[:, t * _D:(t + 1) * _D] = res

    inp0 = inp_at(0)
    hidden = []
    hpre = None
    for i in range(_N):
        if i == 0:
            h = inp0 + _S * br_ref[0:1, :]
        else:
            rv = jax.lax.dot_general(hpre * (1.0 / _WIN), wr_ref[i], _DN,
                                     preferred_element_type=f32)
            h = inp0 + _S * (rv + br_ref[i:i + 1, :])
        hidden.append(h)
        hpre = h if i == 0 else hpre + h
    emit_out(0, hidden)

    for t in range(1, _T):
        inp_t = inp_at(t)
        for i in range(_N):
            h = hidden[i] + inp_t
            cs = _COEF[t - 1][i]
            if cs:
                acc = None
                for (j, c) in cs:
                    term = hidden[j] * (float(c) / (_WIN * _WIN))
                    acc = term if acc is None else acc + term
                rv = jax.lax.dot_general(acc, wr_ref[i], _DN,
                                         preferred_element_type=f32)
                h = h + _S * (rv + br_ref[i:i + 1, :])
            else:
                h = h + _S * br_ref[i:i + 1, :]
            hidden[i] = h
        emit_out(t, hidden)


def kernel(x, W_inp, b_inp, W_out, b_out, W_read, b_read):
    bb, tt, feat = x.shape
    xf = x.reshape(bb, tt * feat)
    out = pl.pallas_call(
        _body,
        out_shape=jax.ShapeDtypeStruct((bb, tt * feat), jnp.float32),
    )(xf, W_inp, b_inp.reshape(1, -1), W_read, b_read,
      W_out, b_out.reshape(1, -1))
    return out.reshape(bb, tt, feat)


# speedup vs baseline: 1.4869x; 1.2523x over previous
import math

import jax
import jax.numpy as jnp
import numpy as np
from jax.experimental import pallas as pl

_M, _D, _N, _R = 16384, 64, 8, 8
_T, _B = 8, 8
_S = 0.5
_PROBS = [0.7, 0.3, 0.5]
_WIN = 2 * _R + 1


def _coef_plan():
    step = int(_M * ((math.sqrt(5) - 1) / 2))
    ptr = np.zeros(_N, np.float32)
    centers = np.zeros((_T, _N), np.int64)
    for t in range(_T):
        for i in range(_N):
            c = int(np.clip(np.int32(ptr[i]), 0, _M - 1))
            centers[t, i] = c
            jump = np.float32((c + step + i) % _M)
            walk = np.float32((ptr[i] + np.float32(1.0)) % _M)
            p = np.float32(_PROBS[i % 3])
            q = np.float32(1.0 - _PROBS[i % 3])
            ptr[i] = np.float32(p * jump) + np.float32(q * walk)
    offs = np.arange(-_R, _R + 1)
    wins = (centers[:, :, None] + offs[None, None, :]) % _M

    coef = []
    for t in range(1, _T):
        row = []
        for i in range(_N):
            wi = set(wins[t, i].tolist())
            cs = {}
            for j in range(i):
                c = len(wi & set(wins[t, j].tolist()))
                if c:
                    cs[j] = c
            row.append(cs)
        coef.append(row)
    return coef


_COEF = _coef_plan()
_DN = (((1,), (1,)), ((), ()))
_K = 1.0 / (_WIN * _WIN)


def _body(x_ref, wi_ref, bi_ref, wr_ref, br_ref, wo_ref, bo_ref, out_ref):
    f32 = jnp.float32

    def br(i):
        return br_ref[i:i + 1, :]

    xs = jnp.concatenate([x_ref[:, t * _D:(t + 1) * _D] for t in range(_T)],
                         axis=0)
    inp_all = jax.lax.dot_general(xs, wi_ref[:], _DN,
                                  preferred_element_type=f32) + bi_ref[:]
    inp = [inp_all[t * _B:(t + 1) * _B] for t in range(_T)]

    h0 = []
    hpre = None
    for i in range(_N):
        if i == 0:
            h = inp[0] + _S * br(0)
        else:
            rv = jax.lax.dot_general(hpre * (1.0 / _WIN), wr_ref[i], _DN,
                                     preferred_element_type=f32)
            h = inp[0] + _S * (rv + br(i))
        h0.append(h)
        hpre = h if i == 0 else hpre + h

    P = [None] * _T
    P[1] = inp[1]
    for t in range(2, _T):
        P[t] = P[t - 1] + inp[t]

    hs = {}
    for i in (0, 1, 2):
        hs[i] = [h0[i] + P[t] + (t * _S) * br(i) for t in range(1, _T)]

    def feed(t, i, srcs):
        terms = []
        for j in srcs:
            c = _COEF[t - 1][i].get(j, 0)
            if c:
                terms.append(hs[j][t - 1] * (c * _K))
        if not terms:
            return jnp.zeros((_B, _D), f32)
        acc = terms[0]
        for term in terms[1:]:
            acc = acc + term
        return acc

    def tier(i, srcs):
        a = jnp.concatenate([feed(t, i, srcs) for t in range(1, _T)], axis=0)
        rv_all = jax.lax.dot_general(a, wr_ref[i], _DN,
                                     preferred_element_type=f32)
        cum = None
        lst = []
        for t in range(1, _T):
            rv = rv_all[(t - 1) * _B:t * _B]
            cum = rv if cum is None else cum + rv
            lst.append(h0[i] + P[t] + (t * _S) * br(i) + _S * cum)
        hs[i] = lst

    for i in (3, 4, 5):
        tier(i, (i - 3,))
    for i in (6, 7):
        tier(i, (i - 6, i - 3))

    sums = []
    for t in range(_T):
        hrow = [h0[i] for i in range(_N)] if t == 0 else \
               [hs[i][t - 1] for i in range(_N)]
        s = hrow[0]
        for i in range(1, _N):
            s = s + hrow[i]
        sums.append(s)
    big = jnp.concatenate(sums, axis=0) * (1.0 / _N)
    res = jax.lax.dot_general(big, wo_ref[:], _DN,
                              preferred_element_type=f32) + bo_ref[:]
    for t in range(_T):
        out_ref[:, t * _D:(t + 1) * _D] = res[t * _B:(t + 1) * _B]


def kernel(x, W_inp, b_inp, W_out, b_out, W_read, b_read):
    bb, tt, feat = x.shape
    xf = x.reshape(bb, tt * feat)
    out = pl.pallas_call(
        _body,
        out_shape=jax.ShapeDtypeStruct((bb, tt * feat), jnp.float32),
    )(xf, W_inp, b_inp.reshape(1, -1), W_read, b_read,
      W_out, b_out.reshape(1, -1))
    return out.reshape(bb, tt, feat)


# speedup vs baseline: 1.5981x; 1.0748x over previous
import math

import jax
import jax.numpy as jnp
import numpy as np
from jax.experimental import pallas as pl

_M, _D, _N, _R = 16384, 64, 8, 8
_T, _B = 8, 8
_S = 0.5
_PROBS = [0.7, 0.3, 0.5]
_WIN = 2 * _R + 1


def _coef_plan():
    step = int(_M * ((math.sqrt(5) - 1) / 2))
    ptr = np.zeros(_N, np.float32)
    centers = np.zeros((_T, _N), np.int64)
    for t in range(_T):
        for i in range(_N):
            c = int(np.clip(np.int32(ptr[i]), 0, _M - 1))
            centers[t, i] = c
            jump = np.float32((c + step + i) % _M)
            walk = np.float32((ptr[i] + np.float32(1.0)) % _M)
            p = np.float32(_PROBS[i % 3])
            q = np.float32(1.0 - _PROBS[i % 3])
            ptr[i] = np.float32(p * jump) + np.float32(q * walk)
    offs = np.arange(-_R, _R + 1)
    wins = (centers[:, :, None] + offs[None, None, :]) % _M

    coef = []
    for t in range(1, _T):
        row = []
        for i in range(_N):
            wi = set(wins[t, i].tolist())
            cs = {}
            for j in range(i):
                c = len(wi & set(wins[t, j].tolist()))
                if c:
                    cs[j] = c
            row.append(cs)
        coef.append(row)
    return coef


_COEF = _coef_plan()
_DN = (((1,), (1,)), ((), ()))
_K = 1.0 / (_WIN * _WIN)


def _body(x_ref, wi_ref, bi_ref, wr_ref, br_ref, wo_ref, bo_ref, out_ref):
    f32 = jnp.float32

    def br(i):
        return br_ref[i:i + 1, :]

    xs = jnp.concatenate([x_ref[:, t * _D:(t + 1) * _D] for t in range(_T)],
                         axis=0)
    inp_all = jax.lax.dot_general(xs, wi_ref[:], _DN,
                                  preferred_element_type=f32) + bi_ref[:]
    inp = [inp_all[t * _B:(t + 1) * _B] for t in range(_T)]

    rows = jax.lax.broadcasted_iota(jnp.int32, (_D, _D), 0)
    cols = jax.lax.broadcasted_iota(jnp.int32, (_D, _D), 1)
    eye = (rows == cols).astype(f32)
    pmd = (((1,), (0,)), ((), ()))
    maps = [(None if i == 0 else eye + (_S / _WIN) * wr_ref[i],
             inp[0] + _S * br(i)) for i in range(_N)]
    d = 1
    while d < _N:
        nxt = list(maps)
        for i in range(d, _N):
            mf, bf = maps[i - d]
            ms, bs2 = maps[i]
            mc = None if mf is None else jax.lax.dot_general(
                ms, mf, pmd, preferred_element_type=f32)
            bc = jax.lax.dot_general(bf, ms, _DN,
                                     preferred_element_type=f32) + bs2
            nxt[i] = (mc, bc)
        maps = nxt
        d *= 2
    hpre_list = [m[1] for m in maps]
    h0 = [hpre_list[0]] + [hpre_list[i] - hpre_list[i - 1]
                           for i in range(1, _N)]

    P = [None] * _T
    P[1] = inp[1]
    for t in range(2, _T):
        P[t] = P[t - 1] + inp[t]

    hs = {}
    for i in (0, 1, 2):
        hs[i] = [h0[i] + P[t] + (t * _S) * br(i) for t in range(1, _T)]

    def feed(t, i, srcs):
        terms = []
        for j in srcs:
            c = _COEF[t - 1][i].get(j, 0)
            if c:
                terms.append(hs[j][t - 1] * (c * _K))
        if not terms:
            return jnp.zeros((_B, _D), f32)
        acc = terms[0]
        for term in terms[1:]:
            acc = acc + term
        return acc

    def tier(i, srcs):
        a = jnp.concatenate([feed(t, i, srcs) for t in range(1, _T)], axis=0)
        rv_all = jax.lax.dot_general(a, wr_ref[i], _DN,
                                     preferred_element_type=f32)
        cum = None
        lst = []
        for t in range(1, _T):
            rv = rv_all[(t - 1) * _B:t * _B]
            cum = rv if cum is None else cum + rv
            lst.append(h0[i] + P[t] + (t * _S) * br(i) + _S * cum)
        hs[i] = lst

    for i in (3, 4, 5):
        tier(i, (i - 3,))
    for i in (6, 7):
        tier(i, (i - 6, i - 3))

    sums = []
    for t in range(_T):
        hrow = [h0[i] for i in range(_N)] if t == 0 else \
               [hs[i][t - 1] for i in range(_N)]
        s = hrow[0]
        for i in range(1, _N):
            s = s + hrow[i]
        sums.append(s)
    big = jnp.concatenate(sums, axis=0) * (1.0 / _N)
    res = jax.lax.dot_general(big, wo_ref[:], _DN,
                              preferred_element_type=f32) + bo_ref[:]
    for t in range(_T):
        out_ref[:, t * _D:(t + 1) * _D] = res[t * _B:(t + 1) * _B]


def kernel(x, W_inp, b_inp, W_out, b_out, W_read, b_read):
    bb, tt, feat = x.shape
    xf = x.reshape(bb, tt * feat)
    out = pl.pallas_call(
        _body,
        out_shape=jax.ShapeDtypeStruct((bb, tt * feat), jnp.float32),
    )(xf, W_inp, b_inp.reshape(1, -1), W_read, b_read,
      W_out, b_out.reshape(1, -1))
    return out.reshape(bb, tt, feat)


# speedup vs baseline: 1.6088x; 1.0067x over previous
import math

import jax
import jax.numpy as jnp
import numpy as np
from jax.experimental import pallas as pl

_M, _D, _N, _R = 16384, 64, 8, 8
_T, _B = 8, 8
_S = 0.5
_PROBS = [0.7, 0.3, 0.5]
_WIN = 2 * _R + 1


def _coef_plan():
    step = int(_M * ((math.sqrt(5) - 1) / 2))
    ptr = np.zeros(_N, np.float32)
    centers = np.zeros((_T, _N), np.int64)
    for t in range(_T):
        for i in range(_N):
            c = int(np.clip(np.int32(ptr[i]), 0, _M - 1))
            centers[t, i] = c
            jump = np.float32((c + step + i) % _M)
            walk = np.float32((ptr[i] + np.float32(1.0)) % _M)
            p = np.float32(_PROBS[i % 3])
            q = np.float32(1.0 - _PROBS[i % 3])
            ptr[i] = np.float32(p * jump) + np.float32(q * walk)
    offs = np.arange(-_R, _R + 1)
    wins = (centers[:, :, None] + offs[None, None, :]) % _M

    coef = []
    for t in range(1, _T):
        row = []
        for i in range(_N):
            wi = set(wins[t, i].tolist())
            cs = {}
            for j in range(i):
                c = len(wi & set(wins[t, j].tolist()))
                if c:
                    cs[j] = c
            row.append(cs)
        coef.append(row)
    return coef


_COEF = _coef_plan()
_DN = (((1,), (1,)), ((), ()))
_K = 1.0 / (_WIN * _WIN)


def _body(x_ref, wi_ref, bi_ref, wr_ref, br_ref, wo_ref, bo_ref, out_ref):
    f32 = jnp.float32

    def br(i):
        return br_ref[i:i + 1, :]

    xs = jnp.concatenate([x_ref[:, t * _D:(t + 1) * _D] for t in range(_T)],
                         axis=0)
    inp_all = jax.lax.dot_general(xs, wi_ref[:], _DN,
                                  preferred_element_type=f32) + bi_ref[:]
    inp = [inp_all[t * _B:(t + 1) * _B] for t in range(_T)]

    rows = jax.lax.broadcasted_iota(jnp.int32, (_D, _D), 0)
    cols = jax.lax.broadcasted_iota(jnp.int32, (_D, _D), 1)
    eye = (rows == cols).astype(f32)
    pmd = (((1,), (0,)), ((), ()))
    maps = [(None if i == 0 else eye + (_S / _WIN) * wr_ref[i],
             inp[0] + _S * br(i)) for i in range(_N)]
    d = 1
    while d < _N:
        nxt = list(maps)
        for i in range(d, _N):
            mf, bf = maps[i - d]
            ms, bs2 = maps[i]
            mc = None if mf is None else jax.lax.dot_general(
                ms, mf, pmd, preferred_element_type=f32)
            bc = jax.lax.dot_general(bf, ms, _DN,
                                     preferred_element_type=f32) + bs2
            nxt[i] = (mc, bc)
        maps = nxt
        d *= 2
    hpre_list = [m[1] for m in maps]
    h0 = [hpre_list[0]] + [hpre_list[i] - hpre_list[i - 1]
                           for i in range(1, _N)]

    P = [None] * _T
    P[1] = inp[1]
    for t in range(2, _T):
        P[t] = P[t - 1] + inp[t]

    hs = {}
    for i in (0, 1, 2):
        hs[i] = [h0[i] + P[t] + (t * _S) * br(i) for t in range(1, _T)]

    def feed(t, i, srcs):
        terms = []
        for j in srcs:
            c = _COEF[t - 1][i].get(j, 0)
            if c:
                terms.append(hs[j][t - 1] * (c * _K))
        if not terms:
            return jnp.zeros((_B, _D), f32)
        acc = terms[0]
        for term in terms[1:]:
            acc = acc + term
        return acc

    def tier(i, srcs):
        a = jnp.concatenate([feed(t, i, srcs) for t in range(1, _T)], axis=0)
        rv_all = jax.lax.dot_general(a, wr_ref[i], _DN,
                                     preferred_element_type=f32)
        cum = None
        lst = []
        for t in range(1, _T):
            rv = rv_all[(t - 1) * _B:t * _B]
            cum = rv if cum is None else cum + rv
            lst.append(h0[i] + P[t] + (t * _S) * br(i) + _S * cum)
        hs[i] = lst

    for i in (3, 4, 5):
        tier(i, (i - 3,))

    for i in (6, 7):
        j6, j3 = i - 6, i - 3
        gi = jax.lax.dot_general(wr_ref[i], wr_ref[j3], pmd,
                                 preferred_element_type=f32)
        a1s, a2s = [], []
        cumfeed3 = jnp.zeros((_B, _D), f32)
        for t in range(1, _T):
            c6 = _COEF[t - 1][i].get(j6, 0)
            c3 = _COEF[t - 1][i].get(j3, 0)
            cp = _COEF[t - 1][j3].get(j6, 0)
            if cp:
                cumfeed3 = cumfeed3 + hs[j6][t - 1] * (cp * _K)
            base3 = h0[j3] + P[t] + (t * _S) * br(j3)
            a1s.append(hs[j6][t - 1] * (c6 * _K) + base3 * (c3 * _K))
            a2s.append(cumfeed3 * (c3 * _K * _S))
        rv1_all = jax.lax.dot_general(jnp.concatenate(a1s, axis=0),
                                      wr_ref[i], _DN,
                                      preferred_element_type=f32)
        rv2_all = jax.lax.dot_general(jnp.concatenate(a2s, axis=0),
                                      gi, _DN, preferred_element_type=f32)
        cum = None
        lst = []
        for t in range(1, _T):
            rv = (rv1_all[(t - 1) * _B:t * _B] +
                  rv2_all[(t - 1) * _B:t * _B])
            cum = rv if cum is None else cum + rv
            lst.append(h0[i] + P[t] + (t * _S) * br(i) + _S * cum)
        hs[i] = lst

    sums = []
    for t in range(_T):
        hrow = [h0[i] for i in range(_N)] if t == 0 else \
               [hs[i][t - 1] for i in range(_N)]
        s = hrow[0]
        for i in range(1, _N):
            s = s + hrow[i]
        sums.append(s)
    big = jnp.concatenate(sums, axis=0) * (1.0 / _N)
    res = jax.lax.dot_general(big, wo_ref[:], _DN,
                              preferred_element_type=f32) + bo_ref[:]
    for t in range(_T):
        out_ref[:, t * _D:(t + 1) * _D] = res[t * _B:(t + 1) * _B]


def kernel(x, W_inp, b_inp, W_out, b_out, W_read, b_read):
    bb, tt, feat = x.shape
    xf = x.reshape(bb, tt * feat)
    out = pl.pallas_call(
        _body,
        out_shape=jax.ShapeDtypeStruct((bb, tt * feat), jnp.float32),
    )(xf, W_inp, b_inp.reshape(1, -1), W_read, b_read,
      W_out, b_out.reshape(1, -1))
    return out.reshape(bb, tt, feat)
